# baseline (speedup 1.0000x reference)
import jax
import jax.numpy as jnp
from jax import lax
from jax.experimental import pallas as pl
from jax.experimental.pallas import tpu as pltpu

VOCAB_BLOCK = 2048
NBUF = 5
NW = 3


def _make_body(B, D, V):
    nblocks = pl.cdiv(V, VOCAB_BLOCK)
    starts = [j * VOCAB_BLOCK for j in range(nblocks)]
    widths = [min(VOCAB_BLOCK, V - s) for s in starts]

    def body(ids_ref, table_hbm, w_hbm, b_ref, out_hbm,
             wbuf, wtail, h_vmem, buf_ref, tail_ref,
             gsem, wsems, wtsem, sems, tail_sem):

        def wload(j):
            s, w = starts[j], widths[j]
            if w == VOCAB_BLOCK:
                return pltpu.make_async_copy(
                    w_hbm.at[:, pl.ds(s, w)], wbuf.at[j % NW], wsems.at[j % NW])
            return pltpu.make_async_copy(
                w_hbm.at[:, pl.ds(s, w)], wtail, wtsem)

        def issue(i, _):
            pltpu.make_async_copy(
                table_hbm.at[pl.ds(ids_ref[i], 1), :],
                h_vmem.at[pl.ds(i, 1), :],
                gsem,
            ).start()
            return 0

        lax.fori_loop(0, B, issue, 0)
        for j in range(min(NW, nblocks)):
            wload(j).start()
        pltpu.make_async_copy(table_hbm.at[pl.ds(0, B), :], h_vmem, gsem).wait()

        h1 = jnp.concatenate(
            [h_vmem[...], jnp.ones((B, 1), jnp.float32)], axis=1)

        def make_cp(j):
            s, w = starts[j], widths[j]
            if w == VOCAB_BLOCK:
                src, sem = buf_ref.at[j % NBUF], sems.at[j % NBUF]
            else:
                src, sem = tail_ref, tail_sem
            return pltpu.make_async_copy(src, out_hbm.at[pl.ds(s, w), :], sem)

        for j in range(nblocks):
            s, w = starts[j], widths[j]
            if j >= NBUF and widths[j - NBUF] == VOCAB_BLOCK:
                make_cp(j - NBUF).wait()
            wload(j).wait()
            wslice = wbuf[j % NW] if w == VOCAB_BLOCK else wtail[...]
            wb = jnp.concatenate([wslice, b_ref[:, s:s + w]], axis=0)
            block = lax.dot_general(
                wb, h1,
                dimension_numbers=(((0,), (1,)), ((), ())),
                preferred_element_type=jnp.float32,
            )
            if w == VOCAB_BLOCK:
                buf_ref[j % NBUF] = block
            else:
                tail_ref[...] = block
            make_cp(j).start()
            if j + NW < nblocks:
                wload(j + NW).start()
        for j in range(max(0, nblocks - NBUF), nblocks):
            make_cp(j).wait()

    return body


def kernel(input_ids, embed_table, W, b):
    V, D = embed_table.shape
    B = input_ids.shape[0]
    b2 = b.reshape(1, V)
    vtail = V % VOCAB_BLOCK or VOCAB_BLOCK
    outT = pl.pallas_call(
        _make_body(B, D, V),
        in_specs=[
            pl.BlockSpec(memory_space=pltpu.SMEM),
            pl.BlockSpec(memory_space=pl.ANY),
            pl.BlockSpec(memory_space=pl.ANY),
            pl.BlockSpec(memory_space=pltpu.VMEM),
        ],
        out_specs=pl.BlockSpec(memory_space=pl.ANY),
        out_shape=jax.ShapeDtypeStruct((V, B), jnp.float32),
        scratch_shapes=[
            pltpu.VMEM((NW, D, VOCAB_BLOCK), jnp.float32),
            pltpu.VMEM((D, vtail), jnp.float32),
            pltpu.VMEM((B, D), jnp.float32),
            pltpu.VMEM((NBUF, VOCAB_BLOCK, B), jnp.float32),
            pltpu.VMEM((vtail, B), jnp.float32),
            pltpu.SemaphoreType.DMA,
            pltpu.SemaphoreType.DMA((NW,)),
            pltpu.SemaphoreType.DMA,
            pltpu.SemaphoreType.DMA((NBUF,)),
            pltpu.SemaphoreType.DMA,
        ],
        compiler_params=pltpu.CompilerParams(
            vmem_limit_bytes=100 * 1024 * 1024,
        ),
    )(input_ids, embed_table, W, b2)
    return outT.T


# speedup vs baseline: 1.0259x; 1.0259x over previous
import jax
import jax.numpy as jnp
from jax import lax
from jax.experimental import pallas as pl
from jax.experimental.pallas import tpu as pltpu

VOCAB_BLOCK = 1024
NBUF = 6


def _make_body(B, D, V):
    nblocks = pl.cdiv(V, VOCAB_BLOCK)
    starts = [j * VOCAB_BLOCK for j in range(nblocks)]
    widths = [min(VOCAB_BLOCK, V - s) for s in starts]

    def body(ids_ref, table_hbm, w_hbm, b_ref, out_hbm,
             w_vmem, h_vmem, buf_ref, tail_ref, wsem, gsem, sems, tail_sem):
        wcp = pltpu.make_async_copy(w_hbm, w_vmem, wsem)
        wcp.start()

        def issue(i, _):
            pltpu.make_async_copy(
                table_hbm.at[pl.ds(ids_ref[i], 1), :],
                h_vmem.at[pl.ds(i, 1), :],
                gsem,
            ).start()
            return 0

        lax.fori_loop(0, B, issue, 0)
        pltpu.make_async_copy(table_hbm.at[pl.ds(0, B), :], h_vmem, gsem).wait()
        wcp.wait()

        h1 = jnp.concatenate(
            [h_vmem[...], jnp.ones((B, 1), jnp.float32)], axis=1)

        def make_cp(j):
            s, w = starts[j], widths[j]
            if w == VOCAB_BLOCK:
                src, sem = buf_ref.at[j % NBUF], sems.at[j % NBUF]
            else:
                src, sem = tail_ref, tail_sem
            return pltpu.make_async_copy(src, out_hbm.at[pl.ds(s, w), :], sem)

        for j in range(nblocks):
            s, w = starts[j], widths[j]
            if j >= NBUF and widths[j - NBUF] == VOCAB_BLOCK:
                make_cp(j - NBUF).wait()
            wb = jnp.concatenate(
                [w_vmem[:, s:s + w], b_ref[:, s:s + w]], axis=0)
            block = lax.dot_general(
                wb, h1,
                dimension_numbers=(((0,), (1,)), ((), ())),
                preferred_element_type=jnp.float32,
            )
            if w == VOCAB_BLOCK:
                buf_ref[j % NBUF] = block
            else:
                tail_ref[...] = block
            make_cp(j).start()
        for j in range(max(0, nblocks - NBUF), nblocks):
            make_cp(j).wait()

    return body


def kernel(input_ids, embed_table, W, b):
    V, D = embed_table.shape
    B = input_ids.shape[0]
    b2 = b.reshape(1, V)
    outT = pl.pallas_call(
        _make_body(B, D, V),
        in_specs=[
            pl.BlockSpec(memory_space=pltpu.SMEM),
            pl.BlockSpec(memory_space=pl.ANY),
            pl.BlockSpec(memory_space=pl.ANY),
            pl.BlockSpec(memory_space=pltpu.VMEM),
        ],
        out_specs=pl.BlockSpec(memory_space=pl.ANY),
        out_shape=jax.ShapeDtypeStruct((V, B), jnp.float32),
        scratch_shapes=[
            pltpu.VMEM((D, V), jnp.float32),
            pltpu.VMEM((B, D), jnp.float32),
            pltpu.VMEM((NBUF, VOCAB_BLOCK, B), jnp.float32),
            pltpu.VMEM((V % VOCAB_BLOCK or VOCAB_BLOCK, B), jnp.float32),
            pltpu.SemaphoreType.DMA,
            pltpu.SemaphoreType.DMA,
            pltpu.SemaphoreType.DMA((NBUF,)),
            pltpu.SemaphoreType.DMA,
        ],
        compiler_params=pltpu.CompilerParams(
            vmem_limit_bytes=100 * 1024 * 1024,
        ),
    )(input_ids, embed_table, W, b2)
    return outT.T
